# baseline (speedup 1.0000x reference)
import jax
import jax.numpy as jnp
from jax.experimental import pallas as pl

_NT = 7
_N = 50000
_D = 128
_BLK = 2000


def _combine_body(x_ref, g_ref, ws_ref, wn_ref, b_ref, o_ref):
    x = x_ref[...]
    acc = jnp.dot(x, jnp.sum(ws_ref[...], axis=0),
                  preferred_element_type=jnp.float32)
    g = g_ref[...]
    for t in range(_NT):
        acc = acc + jnp.dot(g[t], wn_ref[t],
                            preferred_element_type=jnp.float32)
    o_ref[...] = (acc + jnp.sum(b_ref[...], axis=0)) * (1.0 / _NT)


def _combine(x, G, Wself, Wnbr, B):
    n_pad = G.shape[1]
    grid = (_N // _BLK,)
    return pl.pallas_call(
        _combine_body,
        grid=grid,
        in_specs=[
            pl.BlockSpec((_BLK, _D), lambda i: (i, 0)),
            pl.BlockSpec((_NT, _BLK, _D), lambda i: (0, i, 0)),
            pl.BlockSpec((_NT, _D, _D), lambda i: (0, 0, 0)),
            pl.BlockSpec((_NT, _D, _D), lambda i: (0, 0, 0)),
            pl.BlockSpec((_NT, _D), lambda i: (0, 0)),
        ],
        out_specs=pl.BlockSpec((_BLK, _D), lambda i: (i, 0)),
        out_shape=jax.ShapeDtypeStruct((_N, _D), jnp.float32),
    )(x, G, Wself, Wnbr, B)


def kernel(x, edge_index_candidate2candidate, W_self_candidate2candidate, W_nbr_candidate2candidate, b_candidate2candidate, edge_index_candidate2document, W_self_candidate2document, W_nbr_candidate2document, b_candidate2document, edge_index_candidate2entity, W_self_candidate2entity, W_nbr_candidate2entity, b_candidate2entity, edge_index_codocument, W_self_codocument, W_nbr_codocument, b_codocument, edge_index_comention, W_self_comention, W_nbr_comention, b_comention, edge_index_document2entity, W_self_document2entity, W_nbr_document2entity, b_document2entity, edge_index_entity, W_self_entity, W_nbr_entity, b_entity):
    edges = [edge_index_candidate2candidate, edge_index_candidate2document,
             edge_index_candidate2entity, edge_index_codocument,
             edge_index_comention, edge_index_document2entity,
             edge_index_entity]
    Wself = jnp.stack([W_self_candidate2candidate, W_self_candidate2document,
                       W_self_candidate2entity, W_self_codocument,
                       W_self_comention, W_self_document2entity,
                       W_self_entity])
    Wnbr = jnp.stack([W_nbr_candidate2candidate, W_nbr_candidate2document,
                      W_nbr_candidate2entity, W_nbr_codocument,
                      W_nbr_comention, W_nbr_document2entity,
                      W_nbr_entity])
    B = jnp.stack([b_candidate2candidate, b_candidate2document,
                   b_candidate2entity, b_codocument, b_comention,
                   b_document2entity, b_entity])

    gs = []
    for e in edges:
        msg = jnp.take(x, e[0], axis=0)
        gs.append(jax.ops.segment_sum(msg, e[1], num_segments=_N))
    G = jnp.stack(gs)

    return _combine(x, G, Wself, Wnbr, B)


# speedup vs baseline: 3.1243x; 3.1243x over previous
import functools

import jax
import jax.numpy as jnp
from jax import lax
from jax.experimental import pallas as pl
from jax.experimental.pallas import tpu as pltpu
from jax.experimental.pallas import tpu_sc as plsc

_NT = 7
_N = 50000
_D = 128
_E = 80000
_BLK = 2000

_NSUB = 16
_EPS = _E // _NSUB
_EPS_PAD = _EPS + 16
_NVREG = (_EPS + 15) // 16
_NPASS = 3
_RSEG = 8448
_CORE_ROWS = _NPASS * _RSEG
_NPAD = 2 * _CORE_ROWS
_GCH = 128
_LROWS = 41
_RPS = _RSEG // _NSUB
_ZROWS = 64


def _seg_sums(x, src, dst, zeros):
    mesh = plsc.VectorSubcoreMesh(core_axis_name="c", subcore_axis_name="s")

    @functools.partial(
        pl.kernel,
        mesh=mesh,
        compiler_params=pltpu.CompilerParams(needs_layout_passes=False),
        out_type=jax.ShapeDtypeStruct((_NT, _NPAD, _D), jnp.float32),
        scratch_types=[
            pltpu.VMEM((_EPS_PAD,), jnp.int32),
            pltpu.VMEM((_EPS_PAD,), jnp.int32),
            pltpu.VMEM((_LROWS, _GCH), jnp.int32),
            pltpu.VMEM((_LROWS, _GCH), jnp.int32),
            pltpu.VMEM((_GCH, _D), jnp.float32),
            pltpu.VMEM((_ZROWS, _D), jnp.float32),
            pltpu.VMEM_SHARED((_RSEG + 16, _D), jnp.float32),
            pltpu.SemaphoreType.DMA,
        ],
    )
    def body(src_hbm, dst_hbm, x_hbm, z_hbm, g_hbm,
             src_st, dst_st, ssrc, sdst, rowbuf, zbuf, acc, sem):
        c = lax.axis_index("c")
        s = lax.axis_index("s")
        clo = c * _CORE_ROWS
        ebase = s * _EPS
        lanes = lax.iota(jnp.int32, 16)
        zb = s * _RPS

        def zero_own_rows():
            for k in range(_RPS // _ZROWS):
                pltpu.sync_copy(zbuf, acc.at[pl.ds(zb + k * _ZROWS, _ZROWS)])
            rem = _RPS % _ZROWS
            pltpu.sync_copy(zbuf.at[pl.ds(0, rem)],
                            acc.at[pl.ds(zb + _RPS - rem, rem)])

        pltpu.sync_copy(z_hbm, zbuf)
        zero_own_rows()
        plsc.subcore_barrier()

        def type_body(t, _):
            pltpu.sync_copy(src_hbm.at[pl.ds(t * _E + ebase, _EPS)],
                            src_st.at[pl.ds(0, _EPS)])
            pltpu.sync_copy(dst_hbm.at[pl.ds(t * _E + ebase, _EPS)],
                            dst_st.at[pl.ds(0, _EPS)])

            def pass_body(p, _):
                plo = clo + p * _RSEG

                def scan_body(i, cnt):
                    off = i * 16
                    d = dst_st[pl.ds(off, 16)]
                    sv = src_st[pl.ds(off, 16)]
                    valid = (off + lanes) < _EPS
                    dloc = d - plo
                    m = (dloc >= 0) & (dloc < _RSEG) & valid
                    inc = plsc.cumsum(m.astype(jnp.int32))
                    pos = cnt + inc - 1
                    plsc.store_scatter(ssrc, [pos // _GCH, pos % _GCH],
                                       sv, mask=m)
                    plsc.store_scatter(sdst, [pos // _GCH, pos % _GCH],
                                       dloc, mask=m)
                    return cnt + inc[15]

                cnt = lax.fori_loop(0, _NVREG, scan_body,
                                    jnp.zeros((), jnp.int32))

                for k in range(_GCH // 16):
                    pp = cnt + k * 16 + lanes
                    plsc.store_scatter(ssrc, [pp // _GCH, pp % _GCH], lanes)
                    plsc.store_scatter(sdst, [pp // _GCH, pp % _GCH],
                                       _RSEG + lanes)

                def chunk_body(j, _):
                    pltpu.async_copy(x_hbm.at[ssrc.at[j]], rowbuf, sem).wait()
                    pltpu.sync_copy(rowbuf, acc.at[sdst.at[j]], add=True)
                    return 0

                lax.fori_loop(0, (cnt + _GCH - 1) // _GCH, chunk_body, 0)
                plsc.subcore_barrier()

                pltpu.sync_copy(acc.at[pl.ds(zb, _RPS)],
                                g_hbm.at[t, pl.ds(plo + zb, _RPS)])
                zero_own_rows()
                plsc.subcore_barrier()
                return 0

            lax.fori_loop(0, _NPASS, pass_body, 0)
            return 0

        lax.fori_loop(0, _NT, type_body, 0)

    return body(src, dst, x, zeros)


def _combine_body(x_ref, g_ref, ws_ref, wn_ref, b_ref, o_ref):
    x = x_ref[...]
    acc = jnp.dot(x, jnp.sum(ws_ref[...], axis=0),
                  preferred_element_type=jnp.float32)
    g = g_ref[...]
    for t in range(_NT):
        acc = acc + jnp.dot(g[t], wn_ref[t],
                            preferred_element_type=jnp.float32)
    o_ref[...] = (acc + jnp.sum(b_ref[...], axis=0)) * (1.0 / _NT)


def _combine(x, G, Wself, Wnbr, B):
    grid = (_N // _BLK,)
    return pl.pallas_call(
        _combine_body,
        grid=grid,
        in_specs=[
            pl.BlockSpec((_BLK, _D), lambda i: (i, 0)),
            pl.BlockSpec((_NT, _BLK, _D), lambda i: (0, i, 0)),
            pl.BlockSpec((_NT, _D, _D), lambda i: (0, 0, 0)),
            pl.BlockSpec((_NT, _D, _D), lambda i: (0, 0, 0)),
            pl.BlockSpec((_NT, _D), lambda i: (0, 0)),
        ],
        out_specs=pl.BlockSpec((_BLK, _D), lambda i: (i, 0)),
        out_shape=jax.ShapeDtypeStruct((_N, _D), jnp.float32),
    )(x, G, Wself, Wnbr, B)


def kernel(x, edge_index_candidate2candidate, W_self_candidate2candidate, W_nbr_candidate2candidate, b_candidate2candidate, edge_index_candidate2document, W_self_candidate2document, W_nbr_candidate2document, b_candidate2document, edge_index_candidate2entity, W_self_candidate2entity, W_nbr_candidate2entity, b_candidate2entity, edge_index_codocument, W_self_codocument, W_nbr_codocument, b_codocument, edge_index_comention, W_self_comention, W_nbr_comention, b_comention, edge_index_document2entity, W_self_document2entity, W_nbr_document2entity, b_document2entity, edge_index_entity, W_self_entity, W_nbr_entity, b_entity):
    edges = [edge_index_candidate2candidate, edge_index_candidate2document,
             edge_index_candidate2entity, edge_index_codocument,
             edge_index_comention, edge_index_document2entity,
             edge_index_entity]
    Wself = jnp.stack([W_self_candidate2candidate, W_self_candidate2document,
                       W_self_candidate2entity, W_self_codocument,
                       W_self_comention, W_self_document2entity,
                       W_self_entity])
    Wnbr = jnp.stack([W_nbr_candidate2candidate, W_nbr_candidate2document,
                      W_nbr_candidate2entity, W_nbr_codocument,
                      W_nbr_comention, W_nbr_document2entity,
                      W_nbr_entity])
    B = jnp.stack([b_candidate2candidate, b_candidate2document,
                   b_candidate2entity, b_codocument, b_comention,
                   b_document2entity, b_entity])
    SRC = jnp.concatenate([e[0] for e in edges])
    DST = jnp.concatenate([e[1] for e in edges])
    zeros = jnp.zeros((_ZROWS, _D), jnp.float32)

    G = _seg_sums(x, SRC, DST, zeros)
    return _combine(x, G, Wself, Wnbr, B)


# speedup vs baseline: 3.5918x; 1.1496x over previous
import functools

import jax
import jax.numpy as jnp
from jax import lax
from jax.experimental import pallas as pl
from jax.experimental.pallas import tpu as pltpu
from jax.experimental.pallas import tpu_sc as plsc

_NT = 7
_N = 50000
_D = 128
_E = 80000
_BLK = 2000

_NSUB = 16
_EPS = _E // _NSUB
_EPS_PAD = _EPS + 24
_NVREG2 = (_EPS + 31) // 32
_NPASS = 3
_RSEG = 8448
_CORE_ROWS = _NPASS * _RSEG
_NPAD = 2 * _CORE_ROWS
_GCH = 128
_LROWS = 41
_RPS = _RSEG // _NSUB
_ZROWS = 48


def _seg_sums(x, src, dst, zeros):
    mesh = plsc.VectorSubcoreMesh(core_axis_name="c", subcore_axis_name="s")

    @functools.partial(
        pl.kernel,
        mesh=mesh,
        compiler_params=pltpu.CompilerParams(needs_layout_passes=False),
        out_type=jax.ShapeDtypeStruct((_NT, _NPAD, _D), jnp.float32),
        scratch_types=[
            pltpu.VMEM((_EPS_PAD,), jnp.int32),
            pltpu.VMEM((_EPS_PAD,), jnp.int32),
            pltpu.VMEM((_LROWS, _GCH), jnp.int32),
            pltpu.VMEM((_LROWS, _GCH), jnp.int32),
            pltpu.VMEM((_GCH, _D), jnp.float32),
            pltpu.VMEM((_GCH, _D), jnp.float32),
            pltpu.VMEM((_ZROWS, _D), jnp.float32),
            pltpu.VMEM_SHARED((_RSEG + 16, _D), jnp.float32),
            pltpu.SemaphoreType.DMA,
            pltpu.SemaphoreType.DMA,
        ],
    )
    def body(src_hbm, dst_hbm, x_hbm, z_hbm, g_hbm,
             src_st, dst_st, ssrc, sdst, rowbuf0, rowbuf1, zbuf, acc,
             sem0, sem1):
        c = lax.axis_index("c")
        s = lax.axis_index("s")
        clo = c * _CORE_ROWS
        ebase = s * _EPS
        lanes = lax.iota(jnp.int32, 16)
        zb = s * _RPS

        def zero_own_rows():
            for k in range(_RPS // _ZROWS):
                pltpu.sync_copy(zbuf, acc.at[pl.ds(zb + k * _ZROWS, _ZROWS)])

        pltpu.sync_copy(z_hbm, zbuf)
        zero_own_rows()
        plsc.subcore_barrier()

        def type_body(t, _):
            pltpu.sync_copy(src_hbm.at[pl.ds(t * _E + ebase, _EPS)],
                            src_st.at[pl.ds(0, _EPS)])
            pltpu.sync_copy(dst_hbm.at[pl.ds(t * _E + ebase, _EPS)],
                            dst_st.at[pl.ds(0, _EPS)])

            def pass_body(p, _):
                plo = clo + p * _RSEG

                def scan_body(i, cnt):
                    off = i * 32
                    da = dst_st[pl.ds(off, 16)]
                    sa = src_st[pl.ds(off, 16)]
                    db = dst_st[pl.ds(off + 16, 16)]
                    sb = src_st[pl.ds(off + 16, 16)]
                    va = (off + lanes) < _EPS
                    vb = (off + 16 + lanes) < _EPS
                    dla = da - plo
                    dlb = db - plo
                    ma = (dla >= 0) & (dla < _RSEG) & va
                    mb = (dlb >= 0) & (dlb < _RSEG) & vb
                    ia = plsc.cumsum(ma.astype(jnp.int32))
                    ib = plsc.cumsum(mb.astype(jnp.int32))
                    pa = cnt + ia - 1
                    plsc.store_scatter(ssrc, [pa // _GCH, pa % _GCH],
                                       sa, mask=ma)
                    plsc.store_scatter(sdst, [pa // _GCH, pa % _GCH],
                                       dla, mask=ma)
                    cmid = cnt + ia[15]
                    pb = cmid + ib - 1
                    plsc.store_scatter(ssrc, [pb // _GCH, pb % _GCH],
                                       sb, mask=mb)
                    plsc.store_scatter(sdst, [pb // _GCH, pb % _GCH],
                                       dlb, mask=mb)
                    return cmid + ib[15]

                cnt = lax.fori_loop(0, _NVREG2, scan_body,
                                    jnp.zeros((), jnp.int32))

                for k in range(_GCH // 16):
                    pp = cnt + k * 16 + lanes
                    plsc.store_scatter(ssrc, [pp // _GCH, pp % _GCH], lanes)
                    plsc.store_scatter(sdst, [pp // _GCH, pp % _GCH],
                                       _RSEG + lanes)

                nch = (cnt + _GCH - 1) // _GCH

                @pl.when(nch > 0)
                def _():
                    pltpu.async_copy(x_hbm.at[ssrc.at[0]], rowbuf0, sem0)

                def pair_body(jj, _):
                    for b, bufc, semc, bufn, semn in (
                            (0, rowbuf0, sem0, rowbuf1, sem1),
                            (1, rowbuf1, sem1, rowbuf0, sem0)):
                        j = jj * 2 + b

                        @pl.when(j < nch)
                        def _():
                            pltpu.make_async_copy(
                                x_hbm.at[ssrc.at[j]], bufc, semc).wait()

                            @pl.when(j + 1 < nch)
                            def _():
                                pltpu.async_copy(
                                    x_hbm.at[ssrc.at[j + 1]], bufn, semn)

                            pltpu.sync_copy(bufc, acc.at[sdst.at[j]],
                                            add=True)
                    return 0

                lax.fori_loop(0, (nch + 1) // 2, pair_body, 0)
                plsc.subcore_barrier()

                pltpu.sync_copy(acc.at[pl.ds(zb, _RPS)],
                                g_hbm.at[t, pl.ds(plo + zb, _RPS)])
                zero_own_rows()
                plsc.subcore_barrier()
                return 0

            lax.fori_loop(0, _NPASS, pass_body, 0)
            return 0

        lax.fori_loop(0, _NT, type_body, 0)

    return body(src, dst, x, zeros)


def _combine_body(x_ref, g_ref, ws_ref, wn_ref, b_ref, o_ref):
    x = x_ref[...]
    acc = jnp.dot(x, jnp.sum(ws_ref[...], axis=0),
                  preferred_element_type=jnp.float32)
    g = g_ref[...]
    for t in range(_NT):
        acc = acc + jnp.dot(g[t], wn_ref[t],
                            preferred_element_type=jnp.float32)
    o_ref[...] = (acc + jnp.sum(b_ref[...], axis=0)) * (1.0 / _NT)


def _combine(x, G, Wself, Wnbr, B):
    grid = (_N // _BLK,)
    return pl.pallas_call(
        _combine_body,
        grid=grid,
        in_specs=[
            pl.BlockSpec((_BLK, _D), lambda i: (i, 0)),
            pl.BlockSpec((_NT, _BLK, _D), lambda i: (0, i, 0)),
            pl.BlockSpec((_NT, _D, _D), lambda i: (0, 0, 0)),
            pl.BlockSpec((_NT, _D, _D), lambda i: (0, 0, 0)),
            pl.BlockSpec((_NT, _D), lambda i: (0, 0)),
        ],
        out_specs=pl.BlockSpec((_BLK, _D), lambda i: (i, 0)),
        out_shape=jax.ShapeDtypeStruct((_N, _D), jnp.float32),
    )(x, G, Wself, Wnbr, B)


def kernel(x, edge_index_candidate2candidate, W_self_candidate2candidate, W_nbr_candidate2candidate, b_candidate2candidate, edge_index_candidate2document, W_self_candidate2document, W_nbr_candidate2document, b_candidate2document, edge_index_candidate2entity, W_self_candidate2entity, W_nbr_candidate2entity, b_candidate2entity, edge_index_codocument, W_self_codocument, W_nbr_codocument, b_codocument, edge_index_comention, W_self_comention, W_nbr_comention, b_comention, edge_index_document2entity, W_self_document2entity, W_nbr_document2entity, b_document2entity, edge_index_entity, W_self_entity, W_nbr_entity, b_entity):
    edges = [edge_index_candidate2candidate, edge_index_candidate2document,
             edge_index_candidate2entity, edge_index_codocument,
             edge_index_comention, edge_index_document2entity,
             edge_index_entity]
    Wself = jnp.stack([W_self_candidate2candidate, W_self_candidate2document,
                       W_self_candidate2entity, W_self_codocument,
                       W_self_comention, W_self_document2entity,
                       W_self_entity])
    Wnbr = jnp.stack([W_nbr_candidate2candidate, W_nbr_candidate2document,
                      W_nbr_candidate2entity, W_nbr_codocument,
                      W_nbr_comention, W_nbr_document2entity,
                      W_nbr_entity])
    B = jnp.stack([b_candidate2candidate, b_candidate2document,
                   b_candidate2entity, b_codocument, b_comention,
                   b_document2entity, b_entity])
    SRC = jnp.concatenate([e[0] for e in edges])
    DST = jnp.concatenate([e[1] for e in edges])
    zeros = jnp.zeros((_ZROWS, _D), jnp.float32)

    G = _seg_sums(x, SRC, DST, zeros)
    return _combine(x, G, Wself, Wnbr, B)


# speedup vs baseline: 3.7636x; 1.0478x over previous
import functools

import jax
import jax.numpy as jnp
from jax import lax
from jax.experimental import pallas as pl
from jax.experimental.pallas import tpu as pltpu
from jax.experimental.pallas import tpu_sc as plsc

_NT = 7
_N = 50000
_D = 128
_E = 80000
_BLK = 2000

_NSUB = 16
_EPS = _E // _NSUB
_EPS_PAD = _EPS + 24
_NVREG2 = (_EPS + 31) // 32
_NPASS = 3
_RSEG = 8448
_CORE_ROWS = _NPASS * _RSEG
_NPAD = 2 * _CORE_ROWS
_GCH = 128
_LROWS = 41
_RPS = _RSEG // _NSUB


def _seg_sums(x, src, dst, zeros):
    mesh = plsc.VectorSubcoreMesh(core_axis_name="c", subcore_axis_name="s")

    @functools.partial(
        pl.kernel,
        mesh=mesh,
        compiler_params=pltpu.CompilerParams(needs_layout_passes=False),
        out_type=jax.ShapeDtypeStruct((_NT, _NPAD, _D), jnp.float32),
        scratch_types=[
            pltpu.VMEM((_EPS_PAD,), jnp.int32),
            pltpu.VMEM((_EPS_PAD,), jnp.int32),
            pltpu.VMEM((_LROWS, _GCH), jnp.int32),
            pltpu.VMEM((_LROWS, _GCH), jnp.int32),
            pltpu.VMEM((_GCH, _D), jnp.float32),
            pltpu.VMEM((_GCH, _D), jnp.float32),
            pltpu.VMEM_SHARED((_RSEG + 16, _D), jnp.float32),
            pltpu.SemaphoreType.DMA,
            pltpu.SemaphoreType.DMA,
            pltpu.SemaphoreType.DMA,
            pltpu.SemaphoreType.DMA,
        ],
    )
    def body(src_hbm, dst_hbm, x_hbm, z_hbm, g_hbm,
             src_st, dst_st, ssrc, sdst, rowbuf0, rowbuf1, acc,
             sem0, sem1, semw, semz):
        c = lax.axis_index("c")
        s = lax.axis_index("s")
        clo = c * _CORE_ROWS
        ebase = s * _EPS
        lanes = lax.iota(jnp.int32, 16)
        zb = s * _RPS

        pltpu.async_copy(z_hbm.at[pl.ds(zb, _RPS)],
                         acc.at[pl.ds(zb, _RPS)], semz).wait()
        plsc.subcore_barrier()

        def scan(plo):
            def scan_body(i, cnt):
                off = i * 32
                da = dst_st[pl.ds(off, 16)]
                sa = src_st[pl.ds(off, 16)]
                db = dst_st[pl.ds(off + 16, 16)]
                sb = src_st[pl.ds(off + 16, 16)]
                va = (off + lanes) < _EPS
                vb = (off + 16 + lanes) < _EPS
                dla = da - plo
                dlb = db - plo
                ma = (dla >= 0) & (dla < _RSEG) & va
                mb = (dlb >= 0) & (dlb < _RSEG) & vb
                ia = plsc.cumsum(ma.astype(jnp.int32))
                ib = plsc.cumsum(mb.astype(jnp.int32))
                pa = cnt + ia - 1
                plsc.store_scatter(ssrc, [pa // _GCH, pa % _GCH],
                                   sa, mask=ma)
                plsc.store_scatter(sdst, [pa // _GCH, pa % _GCH],
                                   dla, mask=ma)
                cmid = cnt + ia[15]
                pb = cmid + ib - 1
                plsc.store_scatter(ssrc, [pb // _GCH, pb % _GCH],
                                   sb, mask=mb)
                plsc.store_scatter(sdst, [pb // _GCH, pb % _GCH],
                                   dlb, mask=mb)
                return cmid + ib[15]

            cnt = lax.fori_loop(0, _NVREG2, scan_body,
                                jnp.zeros((), jnp.int32))

            for k in range(_GCH // 16):
                pp = cnt + k * 16 + lanes
                plsc.store_scatter(ssrc, [pp // _GCH, pp % _GCH], lanes)
                plsc.store_scatter(sdst, [pp // _GCH, pp % _GCH],
                                   _RSEG + lanes)
            return cnt

        def chunks(cnt):
            nch = (cnt + _GCH - 1) // _GCH

            @pl.when(nch > 0)
            def _():
                pltpu.async_copy(x_hbm.at[ssrc.at[0]], rowbuf0, sem0)

            def pair_body(jj, _):
                for b, bufc, semc, bufn, semn in (
                        (0, rowbuf0, sem0, rowbuf1, sem1),
                        (1, rowbuf1, sem1, rowbuf0, sem0)):
                    j = jj * 2 + b

                    @pl.when(j < nch)
                    def _():
                        pltpu.make_async_copy(
                            x_hbm.at[ssrc.at[j]], bufc, semc).wait()

                        @pl.when(j + 1 < nch)
                        def _():
                            pltpu.async_copy(
                                x_hbm.at[ssrc.at[j + 1]], bufn, semn)

                        pltpu.sync_copy(bufc, acc.at[sdst.at[j]],
                                        add=True)
                return 0

            lax.fori_loop(0, (nch + 1) // 2, pair_body, 0)

        def type_body(t, _):
            pltpu.sync_copy(src_hbm.at[pl.ds(t * _E + ebase, _EPS)],
                            src_st.at[pl.ds(0, _EPS)])
            pltpu.sync_copy(dst_hbm.at[pl.ds(t * _E + ebase, _EPS)],
                            dst_st.at[pl.ds(0, _EPS)])

            cnt = scan(clo)
            for p in range(_NPASS):
                chunks(cnt)
                plsc.subcore_barrier()
                w = pltpu.async_copy(
                    acc.at[pl.ds(zb, _RPS)],
                    g_hbm.at[t, pl.ds(clo + p * _RSEG + zb, _RPS)], semw)
                if p + 1 < _NPASS:
                    cnt = scan(clo + (p + 1) * _RSEG)
                w.wait()
                pltpu.async_copy(z_hbm.at[pl.ds(zb, _RPS)],
                                 acc.at[pl.ds(zb, _RPS)], semz).wait()
                plsc.subcore_barrier()
            return 0

        lax.fori_loop(0, _NT, type_body, 0)

    return body(src, dst, x, zeros)


def _combine_body(x_ref, g_ref, ws_ref, wn_ref, b_ref, o_ref):
    x = x_ref[...]
    acc = jnp.dot(x, jnp.sum(ws_ref[...], axis=0),
                  preferred_element_type=jnp.float32)
    g = g_ref[...]
    for t in range(_NT):
        acc = acc + jnp.dot(g[t], wn_ref[t],
                            preferred_element_type=jnp.float32)
    o_ref[...] = (acc + jnp.sum(b_ref[...], axis=0)) * (1.0 / _NT)


def _combine(x, G, Wself, Wnbr, B):
    grid = (_N // _BLK,)
    return pl.pallas_call(
        _combine_body,
        grid=grid,
        in_specs=[
            pl.BlockSpec((_BLK, _D), lambda i: (i, 0)),
            pl.BlockSpec((_NT, _BLK, _D), lambda i: (0, i, 0)),
            pl.BlockSpec((_NT, _D, _D), lambda i: (0, 0, 0)),
            pl.BlockSpec((_NT, _D, _D), lambda i: (0, 0, 0)),
            pl.BlockSpec((_NT, _D), lambda i: (0, 0)),
        ],
        out_specs=pl.BlockSpec((_BLK, _D), lambda i: (i, 0)),
        out_shape=jax.ShapeDtypeStruct((_N, _D), jnp.float32),
    )(x, G, Wself, Wnbr, B)


def kernel(x, edge_index_candidate2candidate, W_self_candidate2candidate, W_nbr_candidate2candidate, b_candidate2candidate, edge_index_candidate2document, W_self_candidate2document, W_nbr_candidate2document, b_candidate2document, edge_index_candidate2entity, W_self_candidate2entity, W_nbr_candidate2entity, b_candidate2entity, edge_index_codocument, W_self_codocument, W_nbr_codocument, b_codocument, edge_index_comention, W_self_comention, W_nbr_comention, b_comention, edge_index_document2entity, W_self_document2entity, W_nbr_document2entity, b_document2entity, edge_index_entity, W_self_entity, W_nbr_entity, b_entity):
    edges = [edge_index_candidate2candidate, edge_index_candidate2document,
             edge_index_candidate2entity, edge_index_codocument,
             edge_index_comention, edge_index_document2entity,
             edge_index_entity]
    Wself = jnp.stack([W_self_candidate2candidate, W_self_candidate2document,
                       W_self_candidate2entity, W_self_codocument,
                       W_self_comention, W_self_document2entity,
                       W_self_entity])
    Wnbr = jnp.stack([W_nbr_candidate2candidate, W_nbr_candidate2document,
                      W_nbr_candidate2entity, W_nbr_codocument,
                      W_nbr_comention, W_nbr_document2entity,
                      W_nbr_entity])
    B = jnp.stack([b_candidate2candidate, b_candidate2document,
                   b_candidate2entity, b_codocument, b_comention,
                   b_document2entity, b_entity])
    SRC = jnp.concatenate([e[0] for e in edges])
    DST = jnp.concatenate([e[1] for e in edges])
    zeros = jnp.zeros((_RSEG, _D), jnp.float32)

    G = _seg_sums(x, SRC, DST, zeros)
    return _combine(x, G, Wself, Wnbr, B)
